# baseline (speedup 1.0000x reference)
import jax
import jax.numpy as jnp
from jax.experimental import pallas as pl
from jax.experimental.pallas import tpu as pltpu

_B, _P, _NOBJ, _C = 32, 8732, 20, 81
_PPAD = 8832
_NSUB, _NLANE = 8, 1104
_THRESH = 0.5
_RCH = 4736
_NST = (_B * _P) // _RCH
_RATIO = 3


def _match_kernel(priors_ref, targets_ref, loc_ref, conf_out_ref, stats_out_ref):
    pr = priors_ref[...]
    cx, cy, w, h = pr[0], pr[1], pr[2], pr[3]
    x1 = cx - w * 0.5
    y1 = cy - h * 0.5
    x2 = cx + w * 0.5
    y2 = cy + h * 0.5
    pa = w * h
    pidx = jax.lax.broadcasted_iota(
        jnp.int32, (_NSUB, _NLANE), 0
    ) * _NLANE + jax.lax.broadcasted_iota(jnp.int32, (_NSUB, _NLANE), 1)
    valid = pidx < _P

    bo = jnp.full((_NSUB, _NLANE), -1.0, jnp.float32)
    bi = jnp.zeros((_NSUB, _NLANE), jnp.int32)
    tvals = []
    bps = []
    for t in range(_NOBJ):
        tx1 = targets_ref[0, t, 0]
        ty1 = targets_ref[0, t, 1]
        tx2 = targets_ref[0, t, 2]
        ty2 = targets_ref[0, t, 3]
        tl = targets_ref[0, t, 4]
        tvals.append((tx1, ty1, tx2, ty2, tl))
        ta = (tx2 - tx1) * (ty2 - ty1)
        iw = jnp.maximum(jnp.minimum(x2, tx2) - jnp.maximum(x1, tx1), 0.0)
        ih = jnp.maximum(jnp.minimum(y2, ty2) - jnp.maximum(y1, ty1), 0.0)
        inter = iw * ih
        ov = inter / (ta + pa - inter)
        ov = jnp.where(valid, ov, -1.0)
        mx = jnp.max(ov)
        bp = jnp.min(jnp.where(ov == mx, pidx, jnp.int32(2**30)))
        bps.append(bp)
        upd = ov > bo
        bo = jnp.where(upd, ov, bo)
        bi = jnp.where(upd, t, bi)
    for t in range(_NOBJ):
        m = pidx == bps[t]
        bo = jnp.where(m, 2.0, bo)
        bi = jnp.where(m, t, bi)
    mx1 = jnp.zeros((_NSUB, _NLANE), jnp.float32)
    my1 = jnp.zeros((_NSUB, _NLANE), jnp.float32)
    mx2 = jnp.zeros((_NSUB, _NLANE), jnp.float32)
    my2 = jnp.zeros((_NSUB, _NLANE), jnp.float32)
    lab = jnp.zeros((_NSUB, _NLANE), jnp.float32)
    for t in range(_NOBJ):
        tx1, ty1, tx2, ty2, tl = tvals[t]
        sel = bi == t
        mx1 = jnp.where(sel, tx1, mx1)
        my1 = jnp.where(sel, ty1, my1)
        mx2 = jnp.where(sel, tx2, mx2)
        my2 = jnp.where(sel, ty2, my2)
        lab = jnp.where(sel, tl, lab)
    conf = lab.astype(jnp.int32) + 1
    conf = jnp.where(bo < _THRESH, 0, conf)
    conf = jnp.where(valid, conf, 0)
    gcx = ((mx1 + mx2) * 0.5 - cx) / (0.1 * w)
    gcy = ((my1 + my2) * 0.5 - cy) / (0.1 * h)
    gw = jnp.log((mx2 - mx1) / w) / 0.2
    gh = jnp.log((my2 - my1) / h) / 0.2
    ld = loc_ref[0]
    pos = conf > 0
    lsum = jnp.float32(0.0)
    for row, g in enumerate((gcx, gcy, gw, gh)):
        d = ld[row] - g
        ad = jnp.abs(d)
        sl1 = jnp.where(ad < 1.0, 0.5 * d * d, ad - 0.5)
        lsum = lsum + jnp.sum(jnp.where(pos, sl1, 0.0))
    npos = jnp.sum(jnp.where(pos, 1.0, 0.0))
    conf_out_ref[0] = conf
    lane = jax.lax.broadcasted_iota(jnp.int32, (1, 128), 1)
    stats_out_ref[0] = jnp.where(lane == 0, lsum, jnp.where(lane == 1, npos, 0.0))


def _lse_kernel(x_ref, c_ref, mine_ref, pp_ref):
    x = x_ref[...]
    c = c_ref[...]
    m = jnp.max(x, axis=1, keepdims=True)
    e = jnp.exp(x - m)
    s = jnp.sum(e, axis=1, keepdims=True)
    lse = jnp.log(s) + m
    lanes = jax.lax.broadcasted_iota(jnp.int32, (_RCH, _C), 1)
    g = jnp.sum(jnp.where(lanes == c, x, 0.0), axis=1, keepdims=True)
    ce = lse - g
    posm = c > 0
    mine_ref[...] = jnp.where(posm, 0.0, ce)
    pp = jnp.sum(jnp.where(posm, ce, 0.0))
    lane = jax.lax.broadcasted_iota(jnp.int32, (1, 128), 1)
    pp_ref[0] = jnp.where(lane == 0, pp, 0.0)


def _topk_kernel(mine_ref, stats_ref, pp_ref, out_ref):
    mine = mine_ref[...]
    stats = stats_ref[...].reshape(_B, 128)
    pp = pp_ref[...].reshape(_NST, 128)
    lossl = jnp.sum(stats[:, 0:1])
    nposf = stats[:, 1:2]
    n_total = jnp.sum(nposf)
    k = jnp.minimum(_RATIO * nposf.astype(jnp.int32), _P - 1)
    bits = jax.lax.bitcast_convert_type(mine, jnp.int32)
    lo = jnp.zeros((_B, 1), jnp.int32)
    hi = jnp.full((_B, 1), 0x7F800000, jnp.int32)

    def body(_, lohi):
        lo, hi = lohi
        mid = lo + (hi - lo) // 2
        cnt = jnp.sum((bits >= mid).astype(jnp.int32), axis=1, keepdims=True)
        ok = cnt >= k
        return jnp.where(ok, mid, lo), jnp.where(ok, hi, mid)

    lo, hi = jax.lax.fori_loop(0, 31, body, (lo, hi))
    v = jax.lax.bitcast_convert_type(lo, jnp.float32)
    gt = bits > lo
    cgt = jnp.sum(gt.astype(jnp.int32), axis=1, keepdims=True)
    sgt = jnp.sum(jnp.where(gt, mine, 0.0), axis=1, keepdims=True)
    neg = sgt + (k - cgt).astype(jnp.float32) * v
    lossc = jnp.sum(pp[:, 0:1]) + jnp.sum(neg)
    lane = jax.lax.broadcasted_iota(jnp.int32, (1, 128), 1)
    out_ref[...] = jnp.where(
        lane == 0, lossl / n_total, jnp.where(lane == 1, lossc / n_total, 0.0)
    )


def _run(loc_data, conf_data, priors, targets, interpret=False):
    pad = jnp.broadcast_to(
        jnp.array([0.5, 0.5, 1.0, 1.0], jnp.float32), (_PPAD - _P, 4)
    )
    priors_t = jnp.concatenate([priors, pad], axis=0).T.reshape(4, _NSUB, _NLANE)
    loc3 = jnp.pad(
        loc_data.transpose(0, 2, 1), ((0, 0), (0, 0), (0, _PPAD - _P))
    ).reshape(_B, 4, _NSUB, _NLANE)
    conf_t_full, stats = pl.pallas_call(
        _match_kernel,
        grid=(_B,),
        in_specs=[
            pl.BlockSpec((4, _NSUB, _NLANE), lambda b: (0, 0, 0)),
            pl.BlockSpec((1, _NOBJ, 5), lambda b: (b, 0, 0), memory_space=pltpu.SMEM),
            pl.BlockSpec((1, 4, _NSUB, _NLANE), lambda b: (b, 0, 0, 0)),
        ],
        out_specs=[
            pl.BlockSpec((1, _NSUB, _NLANE), lambda b: (b, 0, 0)),
            pl.BlockSpec((1, 1, 128), lambda b: (b, 0, 0)),
        ],
        out_shape=[
            jax.ShapeDtypeStruct((_B, _NSUB, _NLANE), jnp.int32),
            jax.ShapeDtypeStruct((_B, 1, 128), jnp.float32),
        ],
        interpret=interpret,
    )(priors_t, targets, loc3)
    conf_col = conf_t_full.reshape(_B, _PPAD)[:, :_P].reshape(_B * _P, 1)
    mine, pp = pl.pallas_call(
        _lse_kernel,
        grid=(_NST,),
        in_specs=[
            pl.BlockSpec((_RCH, _C), lambda i: (i, 0)),
            pl.BlockSpec((_RCH, 1), lambda i: (i, 0)),
        ],
        out_specs=[
            pl.BlockSpec((_RCH, 1), lambda i: (i, 0)),
            pl.BlockSpec((1, 1, 128), lambda i: (i, 0, 0)),
        ],
        out_shape=[
            jax.ShapeDtypeStruct((_B * _P, 1), jnp.float32),
            jax.ShapeDtypeStruct((_NST, 1, 128), jnp.float32),
        ],
        interpret=interpret,
    )(conf_data.reshape(_B * _P, _C), conf_col)
    out = pl.pallas_call(
        _topk_kernel,
        grid=(1,),
        in_specs=[
            pl.BlockSpec((_B, _P), lambda i: (0, 0)),
            pl.BlockSpec((_B, 1, 128), lambda i: (0, 0, 0)),
            pl.BlockSpec((_NST, 1, 128), lambda i: (0, 0, 0)),
        ],
        out_specs=pl.BlockSpec((1, 128), lambda i: (0, 0)),
        out_shape=jax.ShapeDtypeStruct((1, 128), jnp.float32),
        interpret=interpret,
    )(mine.reshape(_B, _P), stats, pp)
    return out[0, 0], out[0, 1]


def kernel(loc_data, conf_data, priors, targets):
    return _run(loc_data, conf_data, priors, targets)


# speedup vs baseline: 2.9285x; 2.9285x over previous
import jax
import jax.numpy as jnp
from jax.experimental import pallas as pl
from jax.experimental.pallas import tpu as pltpu

_B, _P, _NOBJ, _C = 32, 8732, 20, 81
_PPAD = 8832
_NSUB, _NLANE = 8, 1104
_THRESH = 0.5
_RCH = 4736
_NST = (_B * _P) // _RCH
_RATIO = 3


def _match_kernel(priors_ref, targets_ref, loc_ref, conf_out_ref, stats_out_ref):
    pr = priors_ref[...]
    cx, cy, w, h = pr[0], pr[1], pr[2], pr[3]
    x1 = cx - w * 0.5
    y1 = cy - h * 0.5
    x2 = cx + w * 0.5
    y2 = cy + h * 0.5
    pa = w * h
    pidx = jax.lax.broadcasted_iota(
        jnp.int32, (_NSUB, _NLANE), 0
    ) * _NLANE + jax.lax.broadcasted_iota(jnp.int32, (_NSUB, _NLANE), 1)
    valid = pidx < _P

    bo = jnp.full((_NSUB, _NLANE), -1.0, jnp.float32)
    bi = jnp.zeros((_NSUB, _NLANE), jnp.int32)
    tvals = []
    bps = []
    for t in range(_NOBJ):
        tx1 = targets_ref[0, t, 0]
        ty1 = targets_ref[0, t, 1]
        tx2 = targets_ref[0, t, 2]
        ty2 = targets_ref[0, t, 3]
        tl = targets_ref[0, t, 4]
        tvals.append((tx1, ty1, tx2, ty2, tl))
        ta = (tx2 - tx1) * (ty2 - ty1)
        iw = jnp.maximum(jnp.minimum(x2, tx2) - jnp.maximum(x1, tx1), 0.0)
        ih = jnp.maximum(jnp.minimum(y2, ty2) - jnp.maximum(y1, ty1), 0.0)
        inter = iw * ih
        ov = inter / (ta + pa - inter)
        ov = jnp.where(valid, ov, -1.0)
        mx = jnp.max(ov)
        bp = jnp.min(jnp.where(ov == mx, pidx, jnp.int32(2**30)))
        bps.append(bp)
        upd = ov > bo
        bo = jnp.where(upd, ov, bo)
        bi = jnp.where(upd, t, bi)
    for t in range(_NOBJ):
        m = pidx == bps[t]
        bo = jnp.where(m, 2.0, bo)
        bi = jnp.where(m, t, bi)
    mx1 = jnp.zeros((_NSUB, _NLANE), jnp.float32)
    my1 = jnp.zeros((_NSUB, _NLANE), jnp.float32)
    mx2 = jnp.zeros((_NSUB, _NLANE), jnp.float32)
    my2 = jnp.zeros((_NSUB, _NLANE), jnp.float32)
    lab = jnp.zeros((_NSUB, _NLANE), jnp.float32)
    for t in range(_NOBJ):
        tx1, ty1, tx2, ty2, tl = tvals[t]
        sel = bi == t
        mx1 = jnp.where(sel, tx1, mx1)
        my1 = jnp.where(sel, ty1, my1)
        mx2 = jnp.where(sel, tx2, mx2)
        my2 = jnp.where(sel, ty2, my2)
        lab = jnp.where(sel, tl, lab)
    conf = lab.astype(jnp.int32) + 1
    conf = jnp.where(bo < _THRESH, 0, conf)
    conf = jnp.where(valid, conf, 0)
    gcx = ((mx1 + mx2) * 0.5 - cx) / (0.1 * w)
    gcy = ((my1 + my2) * 0.5 - cy) / (0.1 * h)
    gw = jnp.log((mx2 - mx1) / w) / 0.2
    gh = jnp.log((my2 - my1) / h) / 0.2
    ld = loc_ref[0]
    pos = conf > 0
    lsum = jnp.float32(0.0)
    for row, g in enumerate((gcx, gcy, gw, gh)):
        d = ld[row] - g
        ad = jnp.abs(d)
        sl1 = jnp.where(ad < 1.0, 0.5 * d * d, ad - 0.5)
        lsum = lsum + jnp.sum(jnp.where(pos, sl1, 0.0))
    npos = jnp.sum(jnp.where(pos, 1.0, 0.0))
    conf_out_ref[0] = conf
    lane = jax.lax.broadcasted_iota(jnp.int32, (1, 128), 1)
    stats_out_ref[0] = jnp.where(lane == 0, lsum, jnp.where(lane == 1, npos, 0.0))


def _lse_kernel(x_ref, c_ref, mine_ref, pp_ref):
    x = x_ref[0]
    cv = c_ref[0]
    m = jnp.max(x, axis=0, keepdims=True)
    e = jnp.exp(x - m)
    s = jnp.sum(e, axis=0, keepdims=True)
    lse = jnp.log(s) + m
    subs = jax.lax.broadcasted_iota(jnp.int32, (_C, _P), 0)
    g = jnp.sum(jnp.where(subs == cv, x, 0.0), axis=0, keepdims=True)
    ce = lse - g
    posm = cv > 0
    mine_ref[0] = jnp.where(posm, 0.0, ce)
    pp = jnp.sum(jnp.where(posm, ce, 0.0))
    lane = jax.lax.broadcasted_iota(jnp.int32, (1, 128), 1)
    pp_ref[0] = jnp.where(lane == 0, pp, 0.0)


def _topk_kernel(mine_ref, stats_ref, pp_ref, out_ref):
    mine = mine_ref[...]
    stats = stats_ref[...].reshape(_B, 128)
    pp = pp_ref[...].reshape(_B, 128)
    lossl = jnp.sum(stats[:, 0:1])
    nposf = stats[:, 1:2]
    n_total = jnp.sum(nposf)
    k = jnp.minimum(_RATIO * nposf.astype(jnp.int32), _P - 1)
    bits = jax.lax.bitcast_convert_type(mine, jnp.int32)
    lo = jnp.zeros((_B, 1), jnp.int32)
    hi = jnp.full((_B, 1), 0x7F800000, jnp.int32)

    def body(_, lohi):
        lo, hi = lohi
        mid = lo + (hi - lo) // 2
        cnt = jnp.sum((bits >= mid).astype(jnp.int32), axis=1, keepdims=True)
        ok = cnt >= k
        return jnp.where(ok, mid, lo), jnp.where(ok, hi, mid)

    lo, hi = jax.lax.fori_loop(0, 31, body, (lo, hi))
    v = jax.lax.bitcast_convert_type(lo, jnp.float32)
    gt = bits > lo
    cgt = jnp.sum(gt.astype(jnp.int32), axis=1, keepdims=True)
    sgt = jnp.sum(jnp.where(gt, mine, 0.0), axis=1, keepdims=True)
    neg = sgt + (k - cgt).astype(jnp.float32) * v
    lossc = jnp.sum(pp[:, 0:1]) + jnp.sum(neg)
    lane = jax.lax.broadcasted_iota(jnp.int32, (1, 128), 1)
    out_ref[...] = jnp.where(
        lane == 0, lossl / n_total, jnp.where(lane == 1, lossc / n_total, 0.0)
    )


def _run(loc_data, conf_data, priors, targets, interpret=False):
    pad = jnp.broadcast_to(
        jnp.array([0.5, 0.5, 1.0, 1.0], jnp.float32), (_PPAD - _P, 4)
    )
    priors_t = jnp.concatenate([priors, pad], axis=0).T.reshape(4, _NSUB, _NLANE)
    loc3 = jnp.pad(
        loc_data.transpose(0, 2, 1), ((0, 0), (0, 0), (0, _PPAD - _P))
    ).reshape(_B, 4, _NSUB, _NLANE)
    conf_t_full, stats = pl.pallas_call(
        _match_kernel,
        grid=(_B,),
        in_specs=[
            pl.BlockSpec((4, _NSUB, _NLANE), lambda b: (0, 0, 0)),
            pl.BlockSpec((1, _NOBJ, 5), lambda b: (b, 0, 0), memory_space=pltpu.SMEM),
            pl.BlockSpec((1, 4, _NSUB, _NLANE), lambda b: (b, 0, 0, 0)),
        ],
        out_specs=[
            pl.BlockSpec((1, _NSUB, _NLANE), lambda b: (b, 0, 0)),
            pl.BlockSpec((1, 1, 128), lambda b: (b, 0, 0)),
        ],
        out_shape=[
            jax.ShapeDtypeStruct((_B, _NSUB, _NLANE), jnp.int32),
            jax.ShapeDtypeStruct((_B, 1, 128), jnp.float32),
        ],
        interpret=interpret,
    )(priors_t, targets, loc3)
    xt = conf_data.transpose(0, 2, 1)
    conf_row = conf_t_full.reshape(_B, _PPAD)[:, None, :_P]
    mine, pp = pl.pallas_call(
        _lse_kernel,
        grid=(_B,),
        in_specs=[
            pl.BlockSpec((1, _C, _P), lambda b: (b, 0, 0)),
            pl.BlockSpec((1, 1, _P), lambda b: (b, 0, 0)),
        ],
        out_specs=[
            pl.BlockSpec((1, 1, _P), lambda b: (b, 0, 0)),
            pl.BlockSpec((1, 1, 128), lambda b: (b, 0, 0)),
        ],
        out_shape=[
            jax.ShapeDtypeStruct((_B, 1, _P), jnp.float32),
            jax.ShapeDtypeStruct((_B, 1, 128), jnp.float32),
        ],
        interpret=interpret,
    )(xt, conf_row)
    out = pl.pallas_call(
        _topk_kernel,
        grid=(1,),
        in_specs=[
            pl.BlockSpec((_B, _P), lambda i: (0, 0)),
            pl.BlockSpec((_B, 1, 128), lambda i: (0, 0, 0)),
            pl.BlockSpec((_B, 1, 128), lambda i: (0, 0, 0)),
        ],
        out_specs=pl.BlockSpec((1, 128), lambda i: (0, 0)),
        out_shape=jax.ShapeDtypeStruct((1, 128), jnp.float32),
        interpret=interpret,
    )(mine.reshape(_B, _P), stats, pp)
    return out[0, 0], out[0, 1]


def kernel(loc_data, conf_data, priors, targets):
    return _run(loc_data, conf_data, priors, targets)


# speedup vs baseline: 2.9305x; 1.0007x over previous
import jax
import jax.numpy as jnp
from jax.experimental import pallas as pl
from jax.experimental.pallas import tpu as pltpu

_B, _P, _NOBJ, _C = 32, 8732, 20, 81
_PPAD = 8832
_NSUB, _NLANE = 8, 1104
_THRESH = 0.5
_RATIO = 3


def _match_kernel(priors_ref, targets_ref, loc_ref, conf_out_ref, stats_out_ref):
    pr = priors_ref[...]
    cx, cy, w, h = pr[0], pr[1], pr[2], pr[3]
    x1 = cx - w * 0.5
    y1 = cy - h * 0.5
    x2 = cx + w * 0.5
    y2 = cy + h * 0.5
    pa = w * h
    pidx = jax.lax.broadcasted_iota(
        jnp.int32, (_NSUB, _NLANE), 0
    ) * _NLANE + jax.lax.broadcasted_iota(jnp.int32, (_NSUB, _NLANE), 1)
    valid = pidx < _P

    bo = jnp.full((_NSUB, _NLANE), -1.0, jnp.float32)
    bi = jnp.zeros((_NSUB, _NLANE), jnp.int32)
    tvals = []
    bps = []
    for t in range(_NOBJ):
        tx1 = targets_ref[0, t, 0]
        ty1 = targets_ref[0, t, 1]
        tx2 = targets_ref[0, t, 2]
        ty2 = targets_ref[0, t, 3]
        tl = targets_ref[0, t, 4]
        tvals.append((tx1, ty1, tx2, ty2, tl))
        ta = (tx2 - tx1) * (ty2 - ty1)
        iw = jnp.maximum(jnp.minimum(x2, tx2) - jnp.maximum(x1, tx1), 0.0)
        ih = jnp.maximum(jnp.minimum(y2, ty2) - jnp.maximum(y1, ty1), 0.0)
        inter = iw * ih
        ov = inter / (ta + pa - inter)
        ov = jnp.where(valid, ov, -1.0)
        mx = jnp.max(ov)
        bp = jnp.min(jnp.where(ov == mx, pidx, jnp.int32(2**30)))
        bps.append(bp)
        upd = ov > bo
        bo = jnp.where(upd, ov, bo)
        bi = jnp.where(upd, t, bi)
    for t in range(_NOBJ):
        m = pidx == bps[t]
        bo = jnp.where(m, 2.0, bo)
        bi = jnp.where(m, t, bi)
    mx1 = jnp.zeros((_NSUB, _NLANE), jnp.float32)
    my1 = jnp.zeros((_NSUB, _NLANE), jnp.float32)
    mx2 = jnp.zeros((_NSUB, _NLANE), jnp.float32)
    my2 = jnp.zeros((_NSUB, _NLANE), jnp.float32)
    lab = jnp.zeros((_NSUB, _NLANE), jnp.float32)
    for t in range(_NOBJ):
        tx1, ty1, tx2, ty2, tl = tvals[t]
        sel = bi == t
        mx1 = jnp.where(sel, tx1, mx1)
        my1 = jnp.where(sel, ty1, my1)
        mx2 = jnp.where(sel, tx2, mx2)
        my2 = jnp.where(sel, ty2, my2)
        lab = jnp.where(sel, tl, lab)
    conf = lab.astype(jnp.int32) + 1
    conf = jnp.where(bo < _THRESH, 0, conf)
    conf = jnp.where(valid, conf, 0)
    gcx = ((mx1 + mx2) * 0.5 - cx) / (0.1 * w)
    gcy = ((my1 + my2) * 0.5 - cy) / (0.1 * h)
    gw = jnp.log((mx2 - mx1) / w) / 0.2
    gh = jnp.log((my2 - my1) / h) / 0.2
    ld = loc_ref[0]
    pos = conf > 0
    lsum = jnp.float32(0.0)
    for row, g in enumerate((gcx, gcy, gw, gh)):
        d = ld[row] - g
        ad = jnp.abs(d)
        sl1 = jnp.where(ad < 1.0, 0.5 * d * d, ad - 0.5)
        lsum = lsum + jnp.sum(jnp.where(pos, sl1, 0.0))
    npos = jnp.sum(jnp.where(pos, 1.0, 0.0))
    conf_out_ref[0] = conf
    lane = jax.lax.broadcasted_iota(jnp.int32, (1, 128), 1)
    stats_out_ref[0] = jnp.where(lane == 0, lsum, jnp.where(lane == 1, npos, 0.0))


def _lse_kernel(x_ref, c_ref, mine_ref, pp_ref):
    x = x_ref[0]
    cv = c_ref[0]
    m = jnp.max(x, axis=0, keepdims=True)
    e = jnp.exp(x - m)
    s = jnp.sum(e, axis=0, keepdims=True)
    lse = jnp.log(s) + m
    subs = jax.lax.broadcasted_iota(jnp.int32, (_C, _P), 0)
    g = jnp.sum(jnp.where(subs == cv, x, 0.0), axis=0, keepdims=True)
    ce = lse - g
    posm = cv > 0
    mine_ref[0] = jnp.where(posm, 0.0, ce)
    pp = jnp.sum(jnp.where(posm, ce, 0.0))
    lane = jax.lax.broadcasted_iota(jnp.int32, (1, 128), 1)
    pp_ref[0] = jnp.where(lane == 0, pp, 0.0)


def _topk_kernel(mine_ref, stats_ref, pp_ref, out_ref):
    mine = mine_ref[...]
    stats = stats_ref[...].reshape(_B, 128)
    pp = pp_ref[...].reshape(_B, 128)
    lossl = jnp.sum(stats[:, 0:1])
    nposf = stats[:, 1:2]
    n_total = jnp.sum(nposf)
    k = jnp.minimum(_RATIO * nposf.astype(jnp.int32), _P - 1)
    bits = jax.lax.bitcast_convert_type(mine, jnp.int32)
    lo = jnp.zeros((_B, 1), jnp.int32)
    hi = jnp.full((_B, 1), 0x7F800000, jnp.int32)

    def body(_, lohi):
        lo, hi = lohi
        mid = lo + (hi - lo) // 2
        cnt = jnp.sum((bits >= mid).astype(jnp.int32), axis=1, keepdims=True)
        ok = cnt >= k
        return jnp.where(ok, mid, lo), jnp.where(ok, hi, mid)

    lo, hi = jax.lax.fori_loop(0, 31, body, (lo, hi))
    v = jax.lax.bitcast_convert_type(lo, jnp.float32)
    gt = bits > lo
    cgt = jnp.sum(gt.astype(jnp.int32), axis=1, keepdims=True)
    sgt = jnp.sum(jnp.where(gt, mine, 0.0), axis=1, keepdims=True)
    neg = sgt + (k - cgt).astype(jnp.float32) * v
    lossc = jnp.sum(pp[:, 0:1]) + jnp.sum(neg)
    lane = jax.lax.broadcasted_iota(jnp.int32, (1, 128), 1)
    out_ref[...] = jnp.where(
        lane == 0, lossl / n_total, jnp.where(lane == 1, lossc / n_total, 0.0)
    )


def kernel(loc_data, conf_data, priors, targets):
    pad = jnp.broadcast_to(
        jnp.array([0.5, 0.5, 1.0, 1.0], jnp.float32), (_PPAD - _P, 4)
    )
    priors_t = jnp.concatenate([priors, pad], axis=0).T.reshape(4, _NSUB, _NLANE)
    loc3 = jnp.pad(
        loc_data.transpose(0, 2, 1), ((0, 0), (0, 0), (0, _PPAD - _P))
    ).reshape(_B, 4, _NSUB, _NLANE)
    conf_t_full, stats = pl.pallas_call(
        _match_kernel,
        grid=(_B,),
        in_specs=[
            pl.BlockSpec((4, _NSUB, _NLANE), lambda b: (0, 0, 0)),
            pl.BlockSpec((1, _NOBJ, 5), lambda b: (b, 0, 0), memory_space=pltpu.SMEM),
            pl.BlockSpec((1, 4, _NSUB, _NLANE), lambda b: (b, 0, 0, 0)),
        ],
        out_specs=[
            pl.BlockSpec((1, _NSUB, _NLANE), lambda b: (b, 0, 0)),
            pl.BlockSpec((1, 1, 128), lambda b: (b, 0, 0)),
        ],
        out_shape=[
            jax.ShapeDtypeStruct((_B, _NSUB, _NLANE), jnp.int32),
            jax.ShapeDtypeStruct((_B, 1, 128), jnp.float32),
        ],
    )(priors_t, targets, loc3)
    xt = conf_data.transpose(0, 2, 1)
    conf_row = conf_t_full.reshape(_B, _PPAD)[:, None, :_P]
    mine, pp = pl.pallas_call(
        _lse_kernel,
        grid=(_B,),
        in_specs=[
            pl.BlockSpec((1, _C, _P), lambda b: (b, 0, 0)),
            pl.BlockSpec((1, 1, _P), lambda b: (b, 0, 0)),
        ],
        out_specs=[
            pl.BlockSpec((1, 1, _P), lambda b: (b, 0, 0)),
            pl.BlockSpec((1, 1, 128), lambda b: (b, 0, 0)),
        ],
        out_shape=[
            jax.ShapeDtypeStruct((_B, 1, _P), jnp.float32),
            jax.ShapeDtypeStruct((_B, 1, 128), jnp.float32),
        ],
    )(xt, conf_row)
    out = pl.pallas_call(
        _topk_kernel,
        grid=(1,),
        in_specs=[
            pl.BlockSpec((_B, _P), lambda i: (0, 0)),
            pl.BlockSpec((_B, 1, 128), lambda i: (0, 0, 0)),
            pl.BlockSpec((_B, 1, 128), lambda i: (0, 0, 0)),
        ],
        out_specs=pl.BlockSpec((1, 128), lambda i: (0, 0)),
        out_shape=jax.ShapeDtypeStruct((1, 128), jnp.float32),
    )(mine.reshape(_B, _P), stats, pp)
    return out[0, 0], out[0, 1]
